# baseline (device time: 114143 ns/iter reference)
import numpy as np
import jax
import jax.numpy as jnp
from jax import lax
from jax.experimental import pallas as pl
from jax.experimental.pallas import tpu as pltpu

N_DEV = 16
B, SQ, D = 2, 512, 1024
HQ_LOCAL, DH = 8, 128
SCALE = 0.08838834764831843
ROWS = B * SQ
CHUNK = ROWS // N_DEV
CPB = SQ // CHUNK
CW_STEPS = 8


def _rope_tables():
    inv = 1.0 / (10000.0 ** (np.arange(0, DH, 2) / DH))
    pos = np.arange(SQ)[:, None] * inv[None, :]
    cos = np.repeat(np.cos(pos), 2, axis=-1).astype(np.float32)
    sin = np.repeat(np.sin(pos), 2, axis=-1).astype(np.float32)
    cos_t = np.tile(cos, (1, HQ_LOCAL))
    sin_t = np.tile(sin, (1, HQ_LOCAL))
    even = np.tile(np.array([1.0, 0.0], np.float32), D // 2)[None, :]
    odd = 1.0 - even
    return cos_t, sin_t, even, odd


def kernel(x, Wq, Wk, Wv, Wo):
    cos_t, sin_t, even, odd = (jnp.asarray(a) for a in _rope_tables())

    def body(x_ref, wq_ref, wk_ref, wv_ref, wo_ref,
             cos_ref, sin_ref, even_ref, odd_ref, out_ref,
             x_flat, k_scr, v_scr, partial_ref, result_ref, my_bf, seed_bf,
             cw_rs, ccw_rs, cw_ag, ccw_ag,
             cw_rs_s, cw_rs_r, ccw_rs_s, ccw_rs_r,
             cw_ag_s, cw_ag_r, ccw_ag_s, ccw_ag_r):
        me = lax.axis_index("i")
        right = lax.rem(me + 1, N_DEV)
        left = lax.rem(me + N_DEV - 1, N_DEV)

        def cidx(k):
            return lax.rem(me + k + 2 * N_DEV, N_DEV)

        bsem = pltpu.get_barrier_semaphore()
        for nbr in (left, right):
            pl.semaphore_signal(bsem, inc=1, device_id=(nbr,),
                                device_id_type=pl.DeviceIdType.MESH)
        pl.semaphore_wait(bsem, 2)

        cos_v = cos_ref[...]
        sin_v = sin_ref[...]
        even_v = even_ref[...]
        odd_v = odd_ref[...]
        wq_v = wq_ref[...]
        wo_v = wo_ref[...]

        def rot(t, cos_w, sin_w):
            tr = pltpu.roll(t, 1, 1) * odd_v - pltpu.roll(t, D - 1, 1) * even_v
            return t * cos_w + tr * sin_w

        for b in range(B):
            xb = x_ref[b]
            x_flat[pl.ds(b * SQ, SQ), :] = xb
            k_scr[pl.ds(b * SQ, SQ), :] = rot(
                jnp.dot(xb, wk_ref[...], preferred_element_type=jnp.float32),
                cos_v, sin_v)
            v_scr[pl.ds(b * SQ, SQ), :] = jnp.dot(
                xb, wv_ref[...], preferred_element_type=jnp.float32)

        def compute_chunk(off):
            c = cidx(off)
            rows = pl.ds(c * CHUNK, CHUNK)
            b512 = lax.div(c, CPB) * SQ
            s0 = lax.rem(c, CPB) * CHUNK
            xc = x_flat[rows, :]
            cos_c = cos_ref[pl.ds(s0, CHUNK), :]
            sin_c = sin_ref[pl.ds(s0, CHUNK), :]
            qc = rot(jnp.dot(xc, wq_v, preferred_element_type=jnp.float32),
                     cos_c, sin_c)
            kb = k_scr[pl.ds(b512, SQ), :]
            vb = v_scr[pl.ds(b512, SQ), :]
            ctxs = []
            for h in range(HQ_LOCAL):
                sl = slice(h * DH, (h + 1) * DH)
                s = lax.dot_general(qc[:, sl], kb[:, sl],
                                    (((1,), (1,)), ((), ())),
                                    preferred_element_type=jnp.float32) * SCALE
                m = jnp.max(s, axis=1, keepdims=True)
                w = jnp.exp(s - m)
                w = w / jnp.sum(w, axis=1, keepdims=True)
                ctxs.append(jnp.dot(w, vb[:, sl],
                                    preferred_element_type=jnp.float32))
            ctx = jnp.concatenate(ctxs, axis=1)
            partial_ref[rows, :] = jnp.dot(
                ctx, wo_v, preferred_element_type=jnp.float32)

        sends = []

        def send(src_at, dst_buf, idx, sem_s, sem_r, dev):
            rdma = pltpu.make_async_remote_copy(
                src_ref=src_at, dst_ref=dst_buf.at[idx],
                send_sem=sem_s.at[idx], recv_sem=sem_r.at[idx],
                device_id=(dev,), device_id_type=pl.DeviceIdType.MESH)
            rdma.start()
            sends.append(rdma)

        def wait_recv(buf, idx, sem_s, sem_r):
            pltpu.make_async_remote_copy(
                src_ref=buf.at[idx], dst_ref=buf.at[idx],
                send_sem=sem_s.at[idx], recv_sem=sem_r.at[idx],
                device_id=(right,),
                device_id_type=pl.DeviceIdType.MESH).wait_recv()

        def pchunk(k):
            return partial_ref[pl.ds(cidx(k) * CHUNK, CHUNK), :]

        compute_chunk(8)
        seed_bf[0, :, :] = pchunk(8).astype(jnp.bfloat16)
        send(seed_bf.at[0], cw_rs, 0, cw_rs_s, cw_rs_r, right)
        compute_chunk(9)
        seed_bf[1, :, :] = pchunk(9).astype(jnp.bfloat16)
        send(seed_bf.at[1], ccw_rs, 0, ccw_rs_s, ccw_rs_r, left)
        for s in range(CW_STEPS):
            if s < 7:
                compute_chunk(7 - s)
            if s < 6:
                compute_chunk(s - 6)
            if s == 7:
                compute_chunk(0)
            wait_recv(cw_rs, s, cw_rs_s, cw_rs_r)
            if s < 7:
                cw_rs[s, :, :] = (cw_rs[s].astype(jnp.float32)
                                  + pchunk(7 - s)).astype(jnp.bfloat16)
                send(cw_rs.at[s], cw_rs, s + 1, cw_rs_s, cw_rs_r, right)
            if s < 7:
                wait_recv(ccw_rs, s, ccw_rs_s, ccw_rs_r)
                if s < 6:
                    ccw_rs[s, :, :] = (ccw_rs[s].astype(jnp.float32)
                                       + pchunk(s - 6)).astype(jnp.bfloat16)
                    send(ccw_rs.at[s], ccw_rs, s + 1, ccw_rs_s, ccw_rs_r, left)

        red = (cw_rs[7].astype(jnp.float32) + pchunk(0)
               + ccw_rs[6].astype(jnp.float32))
        result_ref[pl.ds(cidx(0) * CHUNK, CHUNK), :] = red
        my_bf[...] = red.astype(jnp.bfloat16)

        send(my_bf, cw_ag, 0, cw_ag_s, cw_ag_r, right)
        send(my_bf, ccw_ag, 0, ccw_ag_s, ccw_ag_r, left)
        for u in range(CW_STEPS):
            wait_recv(cw_ag, u, cw_ag_s, cw_ag_r)
            result_ref[pl.ds(cidx(-1 - u) * CHUNK, CHUNK), :] = (
                cw_ag[u].astype(jnp.float32))
            if u < 7:
                send(cw_ag.at[u], cw_ag, u + 1, cw_ag_s, cw_ag_r, right)
            if u < 7:
                wait_recv(ccw_ag, u, ccw_ag_s, ccw_ag_r)
                result_ref[pl.ds(cidx(1 + u) * CHUNK, CHUNK), :] = (
                    ccw_ag[u].astype(jnp.float32))
                if u < 6:
                    send(ccw_ag.at[u], ccw_ag, u + 1, ccw_ag_s, ccw_ag_r, left)

        for rdma in sends:
            rdma.wait_send()

        out_ref[0] = result_ref[pl.ds(0, SQ), :]
        out_ref[1] = result_ref[pl.ds(SQ, SQ), :]

    return pl.pallas_call(
        body,
        out_shape=jax.ShapeDtypeStruct((B, SQ, D), jnp.float32),
        in_specs=[pl.BlockSpec(memory_space=pltpu.VMEM)] * 9,
        out_specs=pl.BlockSpec(memory_space=pltpu.VMEM),
        scratch_shapes=[
            pltpu.VMEM((ROWS, D), jnp.float32),
            pltpu.VMEM((ROWS, D), jnp.float32),
            pltpu.VMEM((ROWS, D), jnp.float32),
            pltpu.VMEM((ROWS, D), jnp.float32),
            pltpu.VMEM((ROWS, D), jnp.float32),
            pltpu.VMEM((CHUNK, D), jnp.bfloat16),
            pltpu.VMEM((2, CHUNK, D), jnp.bfloat16),
            pltpu.VMEM((CW_STEPS, CHUNK, D), jnp.bfloat16),
            pltpu.VMEM((CW_STEPS, CHUNK, D), jnp.bfloat16),
            pltpu.VMEM((CW_STEPS, CHUNK, D), jnp.bfloat16),
            pltpu.VMEM((CW_STEPS, CHUNK, D), jnp.bfloat16),
            pltpu.SemaphoreType.DMA((CW_STEPS,)),
            pltpu.SemaphoreType.DMA((CW_STEPS,)),
            pltpu.SemaphoreType.DMA((CW_STEPS,)),
            pltpu.SemaphoreType.DMA((CW_STEPS,)),
            pltpu.SemaphoreType.DMA((CW_STEPS,)),
            pltpu.SemaphoreType.DMA((CW_STEPS,)),
            pltpu.SemaphoreType.DMA((CW_STEPS,)),
            pltpu.SemaphoreType.DMA((CW_STEPS,)),
        ],
        compiler_params=pltpu.CompilerParams(
            collective_id=0, vmem_limit_bytes=100 * 1024 * 1024),
    )(x, Wq, Wk, Wv, Wo, cos_t, sin_t, even, odd)


# device time: 88649 ns/iter; 1.2876x vs baseline; 1.2876x over previous
import numpy as np
import jax
import jax.numpy as jnp
from jax import lax
from jax.experimental import pallas as pl
from jax.experimental.pallas import tpu as pltpu

N_DEV = 16
B, SQ, D = 2, 512, 1024
HQ_LOCAL, DH = 8, 128
SCALE = 0.08838834764831843
ROWS = B * SQ
CHUNK = ROWS // N_DEV
CW_STEPS = 8


def _rope_tables():
    inv = 1.0 / (10000.0 ** (np.arange(0, DH, 2) / DH))
    pos = np.arange(SQ)[:, None] * inv[None, :]
    cos = np.repeat(np.cos(pos), 2, axis=-1).astype(np.float32)
    sin = np.repeat(np.sin(pos), 2, axis=-1).astype(np.float32)
    cos_t = np.tile(cos, (1, HQ_LOCAL))
    sin_t = np.tile(sin, (1, HQ_LOCAL))
    even = np.tile(np.array([1.0, 0.0], np.float32), D // 2)[None, :]
    odd = 1.0 - even
    return cos_t, sin_t, even, odd


def kernel(x, Wq, Wk, Wv, Wo):
    cos_t, sin_t, even, odd = (jnp.asarray(a) for a in _rope_tables())

    def body(x_ref, wq_ref, wk_ref, wv_ref, wo_ref,
             cos_ref, sin_ref, even_ref, odd_ref, out_ref,
             partial_ref, my_bf, seed_bf,
             cw_rs, ccw_rs, cw_ag, ccw_ag,
             cw_rs_s, cw_rs_r, ccw_rs_s, ccw_rs_r,
             cw_ag_s, cw_ag_r, ccw_ag_s, ccw_ag_r):
        me = lax.axis_index("i")
        right = lax.rem(me + 1, N_DEV)
        left = lax.rem(me + N_DEV - 1, N_DEV)

        def cidx(k):
            return lax.rem(me + k + 2 * N_DEV, N_DEV)

        bsem = pltpu.get_barrier_semaphore()
        for nbr in (left, right):
            pl.semaphore_signal(bsem, inc=1, device_id=(nbr,),
                                device_id_type=pl.DeviceIdType.MESH)
        pl.semaphore_wait(bsem, 2)

        cos_v = cos_ref[...]
        sin_v = sin_ref[...]
        even_v = even_ref[...]
        odd_v = odd_ref[...]

        def rot(t):
            tr = pltpu.roll(t, 1, 1) * odd_v - pltpu.roll(t, D - 1, 1) * even_v
            return t * cos_v + tr * sin_v

        for b in range(B):
            xb = x_ref[b]
            q = rot(jnp.dot(xb, wq_ref[...], preferred_element_type=jnp.float32))
            k = rot(jnp.dot(xb, wk_ref[...], preferred_element_type=jnp.float32))
            v = jnp.dot(xb, wv_ref[...], preferred_element_type=jnp.float32)
            ctxs = []
            for h in range(HQ_LOCAL):
                sl = slice(h * DH, (h + 1) * DH)
                qh, kh, vh = q[:, sl], k[:, sl], v[:, sl]
                s = lax.dot_general(qh, kh, (((1,), (1,)), ((), ())),
                                    preferred_element_type=jnp.float32) * SCALE
                w = jnp.exp(s)
                denom = jnp.sum(w, axis=1, keepdims=True)
                ctxs.append(jnp.dot(w, vh, preferred_element_type=jnp.float32)
                            / denom)
            ctx = jnp.concatenate(ctxs, axis=1)
            partial_ref[pl.ds(b * SQ, SQ), :] = jnp.dot(
                ctx, wo_ref[...], preferred_element_type=jnp.float32)

        sends = []

        def send(src_at, dst_buf, idx, sem_s, sem_r, dev):
            rdma = pltpu.make_async_remote_copy(
                src_ref=src_at, dst_ref=dst_buf.at[idx],
                send_sem=sem_s.at[idx], recv_sem=sem_r.at[idx],
                device_id=(dev,), device_id_type=pl.DeviceIdType.MESH)
            rdma.start()
            sends.append(rdma)

        def wait_recv(buf, idx, sem_s, sem_r):
            pltpu.make_async_remote_copy(
                src_ref=buf.at[idx], dst_ref=buf.at[idx],
                send_sem=sem_s.at[idx], recv_sem=sem_r.at[idx],
                device_id=(right,),
                device_id_type=pl.DeviceIdType.MESH).wait_recv()

        def pchunk(k):
            return partial_ref[pl.ds(cidx(k) * CHUNK, CHUNK), :]

        def out_store(c, val):
            bi = lax.div(c, CW_STEPS)
            s0 = lax.rem(c, CW_STEPS) * CHUNK
            out_ref[bi, pl.ds(s0, CHUNK), :] = val

        seed_bf[0, :, :] = pchunk(8).astype(jnp.bfloat16)
        seed_bf[1, :, :] = pchunk(9).astype(jnp.bfloat16)
        send(seed_bf.at[0], cw_rs, 0, cw_rs_s, cw_rs_r, right)
        send(seed_bf.at[1], ccw_rs, 0, ccw_rs_s, ccw_rs_r, left)
        for s in range(CW_STEPS):
            wait_recv(cw_rs, s, cw_rs_s, cw_rs_r)
            if s < 7:
                cw_rs[s, :, :] = (cw_rs[s].astype(jnp.float32)
                                  + pchunk(7 - s)).astype(jnp.bfloat16)
                send(cw_rs.at[s], cw_rs, s + 1, cw_rs_s, cw_rs_r, right)
            if s < 7:
                wait_recv(ccw_rs, s, ccw_rs_s, ccw_rs_r)
                if s < 6:
                    ccw_rs[s, :, :] = (ccw_rs[s].astype(jnp.float32)
                                       + pchunk(s - 6)).astype(jnp.bfloat16)
                    send(ccw_rs.at[s], ccw_rs, s + 1, ccw_rs_s, ccw_rs_r, left)

        red = (cw_rs[7].astype(jnp.float32) + pchunk(0)
               + ccw_rs[6].astype(jnp.float32))
        out_store(cidx(0), red)
        my_bf[...] = red.astype(jnp.bfloat16)

        send(my_bf, cw_ag, 0, cw_ag_s, cw_ag_r, right)
        send(my_bf, ccw_ag, 0, ccw_ag_s, ccw_ag_r, left)
        for u in range(CW_STEPS):
            wait_recv(cw_ag, u, cw_ag_s, cw_ag_r)
            if u < 7:
                send(cw_ag.at[u], cw_ag, u + 1, cw_ag_s, cw_ag_r, right)
            out_store(cidx(-1 - u), cw_ag[u].astype(jnp.float32))
            if u < 7:
                wait_recv(ccw_ag, u, ccw_ag_s, ccw_ag_r)
                if u < 6:
                    send(ccw_ag.at[u], ccw_ag, u + 1, ccw_ag_s, ccw_ag_r, left)
                out_store(cidx(1 + u), ccw_ag[u].astype(jnp.float32))

        for rdma in sends:
            rdma.wait_send()

    return pl.pallas_call(
        body,
        out_shape=jax.ShapeDtypeStruct((B, SQ, D), jnp.float32),
        in_specs=[pl.BlockSpec(memory_space=pltpu.VMEM)] * 9,
        out_specs=pl.BlockSpec(memory_space=pltpu.VMEM),
        scratch_shapes=[
            pltpu.VMEM((ROWS, D), jnp.float32),
            pltpu.VMEM((CHUNK, D), jnp.bfloat16),
            pltpu.VMEM((2, CHUNK, D), jnp.bfloat16),
            pltpu.VMEM((CW_STEPS, CHUNK, D), jnp.bfloat16),
            pltpu.VMEM((CW_STEPS, CHUNK, D), jnp.bfloat16),
            pltpu.VMEM((CW_STEPS, CHUNK, D), jnp.bfloat16),
            pltpu.VMEM((CW_STEPS, CHUNK, D), jnp.bfloat16),
            pltpu.SemaphoreType.DMA((CW_STEPS,)),
            pltpu.SemaphoreType.DMA((CW_STEPS,)),
            pltpu.SemaphoreType.DMA((CW_STEPS,)),
            pltpu.SemaphoreType.DMA((CW_STEPS,)),
            pltpu.SemaphoreType.DMA((CW_STEPS,)),
            pltpu.SemaphoreType.DMA((CW_STEPS,)),
            pltpu.SemaphoreType.DMA((CW_STEPS,)),
            pltpu.SemaphoreType.DMA((CW_STEPS,)),
        ],
        compiler_params=pltpu.CompilerParams(
            collective_id=0, vmem_limit_bytes=100 * 1024 * 1024),
    )(x, Wq, Wk, Wv, Wo, cos_t, sin_t, even, odd)


# device time: 84120 ns/iter; 1.3569x vs baseline; 1.0538x over previous
import numpy as np
import jax
import jax.numpy as jnp
from jax import lax
from jax.experimental import pallas as pl
from jax.experimental.pallas import tpu as pltpu

N_DEV = 16
B, SQ, D = 2, 512, 1024
HQ_LOCAL, DH = 8, 128
SCALE = 0.08838834764831843
ROWS = B * SQ
CHUNK = ROWS // N_DEV
CW_STEPS = 8


def _rope_tables():
    inv = 1.0 / (10000.0 ** (np.arange(0, DH, 2) / DH))
    pos = np.arange(SQ)[:, None] * inv[None, :]
    cos = np.repeat(np.cos(pos), 2, axis=-1).astype(np.float32)
    sin = np.repeat(np.sin(pos), 2, axis=-1).astype(np.float32)
    cos_t = np.tile(cos, (1, HQ_LOCAL))
    sin_t = np.tile(sin, (1, HQ_LOCAL))
    even = np.tile(np.array([1.0, 0.0], np.float32), D // 2)[None, :]
    odd = 1.0 - even
    return cos_t, sin_t, even, odd


def kernel(x, Wq, Wk, Wv, Wo):
    cos_t, sin_t, even, odd = (jnp.asarray(a) for a in _rope_tables())

    def body(x_ref, wq_ref, wk_ref, wv_ref, wo_ref,
             cos_ref, sin_ref, even_ref, odd_ref, out_ref,
             partial_ref, my_bf, seed_bf,
             cw_rs, ccw_rs, cw_ag, ccw_ag,
             cw_rs_s, cw_rs_r, ccw_rs_s, ccw_rs_r,
             cw_ag_s, cw_ag_r, ccw_ag_s, ccw_ag_r):
        me = lax.axis_index("i")
        right = lax.rem(me + 1, N_DEV)
        left = lax.rem(me + N_DEV - 1, N_DEV)

        def cidx(k):
            return lax.rem(me + k + 2 * N_DEV, N_DEV)

        bsem = pltpu.get_barrier_semaphore()
        for nbr in (left, right):
            pl.semaphore_signal(bsem, inc=1, device_id=(nbr,),
                                device_id_type=pl.DeviceIdType.MESH)
        pl.semaphore_wait(bsem, 2)

        cos_v = cos_ref[...]
        sin_v = sin_ref[...]
        even_v = even_ref[...]
        odd_v = odd_ref[...]

        def rot(t):
            tr = pltpu.roll(t, 1, 1) * odd_v - pltpu.roll(t, D - 1, 1) * even_v
            return t * cos_v + tr * sin_v

        for b in range(B):
            xb = x_ref[b]
            q = rot(jnp.dot(xb, wq_ref[...], preferred_element_type=jnp.float32))
            k = rot(jnp.dot(xb, wk_ref[...], preferred_element_type=jnp.float32))
            v = jnp.dot(xb, wv_ref[...], preferred_element_type=jnp.float32)
            ctxs = []
            for h in range(HQ_LOCAL):
                sl = slice(h * DH, (h + 1) * DH)
                qh, kh, vh = q[:, sl], k[:, sl], v[:, sl]
                s = lax.dot_general(qh, kh, (((1,), (1,)), ((), ())),
                                    preferred_element_type=jnp.float32) * SCALE
                w = jnp.exp(s)
                denom = jnp.sum(w, axis=1, keepdims=True)
                ctxs.append(jnp.dot(w, vh, preferred_element_type=jnp.float32)
                            / denom)
            ctx = jnp.concatenate(ctxs, axis=1)
            partial_ref[pl.ds(b * SQ, SQ), :] = jnp.dot(
                ctx, wo_ref[...], preferred_element_type=jnp.float32)

        sends = []
        HALF = CHUNK // 2

        def send(src_at, dst_buf, idx, r, sem_s, sem_r, dev):
            rdma = pltpu.make_async_remote_copy(
                src_ref=src_at,
                dst_ref=dst_buf.at[idx, pl.ds(r * HALF, HALF), :],
                send_sem=sem_s.at[idx * 2 + r], recv_sem=sem_r.at[idx * 2 + r],
                device_id=(dev,), device_id_type=pl.DeviceIdType.MESH)
            rdma.start()
            sends.append(rdma)

        def wait_recv(buf, idx, r, sem_s, sem_r):
            pltpu.make_async_remote_copy(
                src_ref=buf.at[idx, pl.ds(r * HALF, HALF), :],
                dst_ref=buf.at[idx, pl.ds(r * HALF, HALF), :],
                send_sem=sem_s.at[idx * 2 + r], recv_sem=sem_r.at[idx * 2 + r],
                device_id=(right,),
                device_id_type=pl.DeviceIdType.MESH).wait_recv()

        def phalf(k, r):
            return partial_ref[pl.ds(cidx(k) * CHUNK + r * HALF, HALF), :]

        def out_store(c, r, val):
            bi = lax.div(c, CW_STEPS)
            s0 = lax.rem(c, CW_STEPS) * CHUNK + r * HALF
            out_ref[bi, pl.ds(s0, HALF), :] = val

        def rs_half(buf, s, r):
            return buf[s, pl.ds(r * HALF, HALF), :]

        for r in range(2):
            seed_bf[0, pl.ds(r * HALF, HALF), :] = phalf(8, r).astype(
                jnp.bfloat16)
            send(seed_bf.at[0, pl.ds(r * HALF, HALF), :], cw_rs, 0, r,
                 cw_rs_s, cw_rs_r, right)
            seed_bf[1, pl.ds(r * HALF, HALF), :] = phalf(9, r).astype(
                jnp.bfloat16)
            send(seed_bf.at[1, pl.ds(r * HALF, HALF), :], ccw_rs, 0, r,
                 ccw_rs_s, ccw_rs_r, left)
        for s in range(CW_STEPS):
            for r in range(2):
                wait_recv(cw_rs, s, r, cw_rs_s, cw_rs_r)
                if s < 7:
                    cw_rs[s, pl.ds(r * HALF, HALF), :] = (
                        rs_half(cw_rs, s, r).astype(jnp.float32)
                        + phalf(7 - s, r)).astype(jnp.bfloat16)
                    send(cw_rs.at[s, pl.ds(r * HALF, HALF), :], cw_rs,
                         s + 1, r, cw_rs_s, cw_rs_r, right)
            if s < 7:
                for r in range(2):
                    wait_recv(ccw_rs, s, r, ccw_rs_s, ccw_rs_r)
                    if s < 6:
                        ccw_rs[s, pl.ds(r * HALF, HALF), :] = (
                            rs_half(ccw_rs, s, r).astype(jnp.float32)
                            + phalf(s - 6, r)).astype(jnp.bfloat16)
                        send(ccw_rs.at[s, pl.ds(r * HALF, HALF), :], ccw_rs,
                             s + 1, r, ccw_rs_s, ccw_rs_r, left)

        for r in range(2):
            red = (rs_half(cw_rs, 7, r).astype(jnp.float32) + phalf(0, r)
                   + rs_half(ccw_rs, 6, r).astype(jnp.float32))
            out_store(cidx(0), r, red)
            my_bf[pl.ds(r * HALF, HALF), :] = red.astype(jnp.bfloat16)
            send(my_bf.at[pl.ds(r * HALF, HALF), :], cw_ag, 0, r,
                 cw_ag_s, cw_ag_r, right)
            send(my_bf.at[pl.ds(r * HALF, HALF), :], ccw_ag, 0, r,
                 ccw_ag_s, ccw_ag_r, left)

        for u in range(CW_STEPS):
            for r in range(2):
                wait_recv(cw_ag, u, r, cw_ag_s, cw_ag_r)
                if u < 7:
                    send(cw_ag.at[u, pl.ds(r * HALF, HALF), :], cw_ag,
                         u + 1, r, cw_ag_s, cw_ag_r, right)
                out_store(cidx(-1 - u), r,
                          rs_half(cw_ag, u, r).astype(jnp.float32))
            if u < 7:
                for r in range(2):
                    wait_recv(ccw_ag, u, r, ccw_ag_s, ccw_ag_r)
                    if u < 6:
                        send(ccw_ag.at[u, pl.ds(r * HALF, HALF), :], ccw_ag,
                             u + 1, r, ccw_ag_s, ccw_ag_r, left)
                    out_store(cidx(1 + u), r,
                              rs_half(ccw_ag, u, r).astype(jnp.float32))

        for rdma in sends:
            rdma.wait_send()

    return pl.pallas_call(
        body,
        out_shape=jax.ShapeDtypeStruct((B, SQ, D), jnp.float32),
        in_specs=[pl.BlockSpec(memory_space=pltpu.VMEM)] * 9,
        out_specs=pl.BlockSpec(memory_space=pltpu.VMEM),
        scratch_shapes=[
            pltpu.VMEM((ROWS, D), jnp.float32),
            pltpu.VMEM((CHUNK, D), jnp.bfloat16),
            pltpu.VMEM((2, CHUNK, D), jnp.bfloat16),
            pltpu.VMEM((CW_STEPS, CHUNK, D), jnp.bfloat16),
            pltpu.VMEM((CW_STEPS, CHUNK, D), jnp.bfloat16),
            pltpu.VMEM((CW_STEPS, CHUNK, D), jnp.bfloat16),
            pltpu.VMEM((CW_STEPS, CHUNK, D), jnp.bfloat16),
            pltpu.SemaphoreType.DMA((CW_STEPS * 2,)),
            pltpu.SemaphoreType.DMA((CW_STEPS * 2,)),
            pltpu.SemaphoreType.DMA((CW_STEPS * 2,)),
            pltpu.SemaphoreType.DMA((CW_STEPS * 2,)),
            pltpu.SemaphoreType.DMA((CW_STEPS * 2,)),
            pltpu.SemaphoreType.DMA((CW_STEPS * 2,)),
            pltpu.SemaphoreType.DMA((CW_STEPS * 2,)),
            pltpu.SemaphoreType.DMA((CW_STEPS * 2,)),
        ],
        compiler_params=pltpu.CompilerParams(
            collective_id=0, vmem_limit_bytes=100 * 1024 * 1024),
    )(x, Wq, Wk, Wv, Wo, cos_t, sin_t, even, odd)
